# baseline (speedup 1.0000x reference)
import jax
import jax.numpy as jnp
from jax.experimental import pallas as pl
from jax.experimental.pallas import tpu as pltpu


def _attn_kernel(x_ref, nat_ref, w_ref, asrc_ref, adst_ref, g_ref,
                 fcb_ref, o_ref):
    n = x_ref.shape[1]
    H = nat_ref.shape[0]
    C = asrc_ref.shape[1]

    xb = x_ref[0, 0, :, :]
    hh = jnp.dot(xb, w_ref[...], preferred_element_type=jnp.float32)

    fc = jnp.zeros((n, C), dtype=jnp.float32)
    for h in range(H):
        hv = hh[:, h * C:(h + 1) * C]
        s = jnp.sum(hv * asrc_ref[h, :][None, :], axis=1, keepdims=True)
        d = jnp.sum(hv * adst_ref[h, :][None, :], axis=1, keepdims=True)
        e = s + d.T
        e = jnp.where(e > 0, e, 0.2 * e)
        e = e + jnp.where(nat_ref[h] > 0.05, 0.0, -1e9)
        m = jnp.max(e, axis=1, keepdims=True)
        p = jnp.exp(e - m)
        att = p / jnp.sum(p, axis=1, keepdims=True)
        out_h = jnp.dot(att, hv, preferred_element_type=jnp.float32)
        fc = fc + jnp.dot(out_h, g_ref[h * C:(h + 1) * C, :],
                          preferred_element_type=jnp.float32)

    fc = fc + fcb_ref[0, :][None, :]
    o_ref[0, 0, :, :] = jnp.maximum(fc, 0.0)


def kernel(x, NATree, W, a_src, a_dst, fc_w, fc_b):
    b, n, t, c = x.shape
    H = a_src.shape[0]
    C = W.shape[1] // H

    rows = jnp.arange(H * C)
    M = (rows[:, None] // H == jnp.arange(C)[None, :]).astype(jnp.float32) / H
    G = M @ fc_w.T
    fc_b2 = fc_b.reshape(1, C)

    xt = x.transpose(0, 2, 1, 3)

    out = pl.pallas_call(
        _attn_kernel,
        grid=(b, t),
        in_specs=[
            pl.BlockSpec((1, 1, n, c), lambda i, j: (i, j, 0, 0)),
            pl.BlockSpec((H, n, n), lambda i, j: (0, 0, 0)),
            pl.BlockSpec((c, H * C), lambda i, j: (0, 0)),
            pl.BlockSpec((H, C), lambda i, j: (0, 0)),
            pl.BlockSpec((H, C), lambda i, j: (0, 0)),
            pl.BlockSpec((H * C, C), lambda i, j: (0, 0)),
            pl.BlockSpec((1, C), lambda i, j: (0, 0)),
        ],
        out_specs=pl.BlockSpec((1, 1, n, C), lambda i, j: (i, j, 0, 0)),
        out_shape=jax.ShapeDtypeStruct((b, t, n, C), jnp.float32),
        compiler_params=pltpu.CompilerParams(
            dimension_semantics=("parallel", "parallel"),
        ),
    )(xt, NATree, W, a_src, a_dst, G, fc_b2)
    return out.transpose(0, 2, 1, 3)


# speedup vs baseline: 1.7979x; 1.7979x over previous
import jax
import jax.numpy as jnp
from jax.experimental import pallas as pl
from jax.experimental.pallas import tpu as pltpu


def _bias_kernel(nat_ref, o_ref):
    o_ref[...] = jnp.where(nat_ref[...] > 0.05, 0.0, -1e9)


def _attn_kernel(x_ref, bias_ref, wa_ref, wg_ref, fcb_ref, o_ref):
    n = x_ref.shape[2]
    H = bias_ref.shape[0]
    C = o_ref.shape[3]

    xb = x_ref[0, 0]
    sdall = jnp.dot(xb, wa_ref[...],
                    preferred_element_type=jnp.float32)
    sdT = sdall.T
    U = jnp.dot(xb, wg_ref[...],
                preferred_element_type=jnp.float32)

    fc = fcb_ref[0][None, :]
    for h in range(H):
        sd = sdall[:, h:h + 1] + sdT[H + h:H + h + 1, :]
        e = jnp.maximum(sd, 0.2 * sd) + bias_ref[h]
        p = jnp.exp(e)
        pu = jnp.dot(p, U[:, h * 128:(h + 1) * 128],
                     preferred_element_type=jnp.float32)
        fc = fc + pu[:, :C] * (1.0 / pu[:, C:C + 1])
    o_ref[0, 0] = jnp.maximum(fc, 0.0)


def kernel(x, NATree, W, a_src, a_dst, fc_w, fc_b):
    b, n, t, c = x.shape
    H = a_src.shape[0]
    C = W.shape[1] // H

    rows = jnp.arange(H * C)
    M = (rows[:, None] // H == jnp.arange(C)[None, :]).astype(jnp.float32) / H
    G = M @ fc_w.T

    WA = jnp.zeros((c + 1, 2 * H), jnp.float32)
    WG = jnp.zeros((c + 1, H * 128), jnp.float32)
    for h in range(H):
        Wh = W[:, h * C:(h + 1) * C]
        WA = WA.at[:c, h].set(Wh @ a_src[h])
        WA = WA.at[:c, H + h].set(Wh @ a_dst[h])
        WG = WG.at[:c, h * 128:h * 128 + C].set(Wh @ G[h * C:(h + 1) * C])
        WG = WG.at[c, h * 128 + C].set(1.0)

    xe = jnp.concatenate(
        [x.transpose(0, 2, 1, 3),
         jnp.ones((b, t, n, 1), jnp.float32)], axis=-1)

    bias = pl.pallas_call(
        _bias_kernel,
        out_shape=jax.ShapeDtypeStruct((H, n, n), jnp.float32),
    )(NATree)

    out = pl.pallas_call(
        _attn_kernel,
        grid=(b, t),
        in_specs=[
            pl.BlockSpec((1, 1, n, c + 1), lambda i, j: (i, j, 0, 0)),
            pl.BlockSpec((H, n, n), lambda i, j: (0, 0, 0)),
            pl.BlockSpec((c + 1, 2 * H), lambda i, j: (0, 0)),
            pl.BlockSpec((c + 1, H * 128), lambda i, j: (0, 0)),
            pl.BlockSpec((1, C), lambda i, j: (0, 0)),
        ],
        out_specs=pl.BlockSpec((1, 1, n, C), lambda i, j: (i, j, 0, 0)),
        out_shape=jax.ShapeDtypeStruct((b, t, n, C), jnp.float32),
        compiler_params=pltpu.CompilerParams(
            dimension_semantics=("parallel", "parallel"),
        ),
    )(xe, bias, WA, WG, fc_b.reshape(1, C))
    return out.transpose(0, 2, 1, 3)


# speedup vs baseline: 1.7986x; 1.0004x over previous
import jax
import jax.numpy as jnp
from jax.experimental import pallas as pl
from jax.experimental.pallas import tpu as pltpu


def _bias_kernel(nat_ref, o_ref):
    o_ref[...] = jnp.where(nat_ref[...] > 0.05, 0.0, -1e9)


def _attn_kernel(x_ref, bias_ref, wa_ref, wg_ref, fcb_ref, o_ref):
    n = x_ref.shape[1]
    t = x_ref.shape[2]
    H = bias_ref.shape[0]
    C = o_ref.shape[3]

    ones_col = jnp.ones((n, 1), jnp.float32)
    for tt in range(t):
        xb = jnp.concatenate([x_ref[0, :, tt, :], ones_col], axis=1)
        sdall = jnp.dot(xb, wa_ref[...],
                        preferred_element_type=jnp.float32)
        sdT = sdall.T
        U = jnp.dot(xb, wg_ref[...],
                    preferred_element_type=jnp.float32)

        fc = fcb_ref[0][None, :]
        for h in range(H):
            sd = sdall[:, h:h + 1] + sdT[H + h:H + h + 1, :]
            e = jnp.maximum(sd, 0.2 * sd) + bias_ref[h]
            p = jnp.exp(e)
            pu = jnp.dot(p, U[:, h * 128:(h + 1) * 128],
                         preferred_element_type=jnp.float32)
            fc = fc + pu[:, :C] * (1.0 / pu[:, C:C + 1])
        o_ref[0, :, tt, :] = jnp.maximum(fc, 0.0)


def kernel(x, NATree, W, a_src, a_dst, fc_w, fc_b):
    b, n, t, c = x.shape
    H = a_src.shape[0]
    C = W.shape[1] // H

    rows = jnp.arange(H * C)
    M = (rows[:, None] // H == jnp.arange(C)[None, :]).astype(jnp.float32) / H
    G = M @ fc_w.T

    WA = jnp.zeros((c + 1, 2 * H), jnp.float32)
    WG = jnp.zeros((c + 1, H * 128), jnp.float32)
    for h in range(H):
        Wh = W[:, h * C:(h + 1) * C]
        WA = WA.at[:c, h].set(Wh @ a_src[h])
        WA = WA.at[:c, H + h].set(Wh @ a_dst[h])
        WG = WG.at[:c, h * 128:h * 128 + C].set(Wh @ G[h * C:(h + 1) * C])
        WG = WG.at[c, h * 128 + C].set(1.0)

    bias = pl.pallas_call(
        _bias_kernel,
        out_shape=jax.ShapeDtypeStruct((H, n, n), jnp.float32),
    )(NATree)

    out = pl.pallas_call(
        _attn_kernel,
        grid=(b,),
        in_specs=[
            pl.BlockSpec((1, n, t, c), lambda i: (i, 0, 0, 0)),
            pl.BlockSpec((H, n, n), lambda i: (0, 0, 0)),
            pl.BlockSpec((c + 1, 2 * H), lambda i: (0, 0)),
            pl.BlockSpec((c + 1, H * 128), lambda i: (0, 0)),
            pl.BlockSpec((1, C), lambda i: (0, 0)),
        ],
        out_specs=pl.BlockSpec((1, n, t, C), lambda i: (i, 0, 0, 0)),
        out_shape=jax.ShapeDtypeStruct((b, n, t, C), jnp.float32),
        compiler_params=pltpu.CompilerParams(
            dimension_semantics=("parallel",),
        ),
    )(x, bias, WA, WG, fc_b.reshape(1, C))
    return out


# speedup vs baseline: 2.6846x; 1.4926x over previous
import jax
import jax.numpy as jnp
from jax.experimental import pallas as pl
from jax.experimental.pallas import tpu as pltpu


def _prep_kernel(nat_ref, w_ref, asrc_ref, adst_ref, fcw_ref,
                 bias_ref, wa_ref, wg_ref):
    H, n, _ = nat_ref.shape
    c = w_ref.shape[0]
    C = fcw_ref.shape[0]

    bias_ref[...] = jnp.where(nat_ref[...] > 0.05, 0.0, -1e9)

    k_i = jax.lax.broadcasted_iota(jnp.int32, (H * C, C), 0)
    c_i = jax.lax.broadcasted_iota(jnp.int32, (H * C, C), 1)
    M = jnp.where(k_i // H == c_i, 1.0 / H, 0.0)
    G = jnp.dot(M, fcw_ref[...].T, preferred_element_type=jnp.float32)

    wa_cols = []
    wg_blocks = []
    for h in range(H):
        Wh = w_ref[:, h * C:(h + 1) * C]
        wa_cols.append(jnp.dot(Wh, asrc_ref[h, :][:, None],
                               preferred_element_type=jnp.float32))
        wg_blocks.append(jnp.dot(Wh, G[h * C:(h + 1) * C, :],
                                 preferred_element_type=jnp.float32))
        wg_blocks.append(jnp.zeros((c, 128 - C), jnp.float32))
    for h in range(H):
        Wh = w_ref[:, h * C:(h + 1) * C]
        wa_cols.append(jnp.dot(Wh, adst_ref[h, :][:, None],
                               preferred_element_type=jnp.float32))
    wa_ref[:c, :] = jnp.concatenate(wa_cols, axis=1)
    wa_ref[c:, :] = jnp.zeros((1, 2 * H), jnp.float32)

    wg_ref[:c, :] = jnp.concatenate(wg_blocks, axis=1)
    lane = jax.lax.broadcasted_iota(jnp.int32, (1, H * 128), 1)
    wg_ref[c:, :] = jnp.where(lane % 128 == C, 1.0, 0.0)


def _attn_kernel(x_ref, bias_ref, wa_ref, wg_ref, fcb_ref, o_ref):
    n = x_ref.shape[1]
    t = x_ref.shape[2]
    H = bias_ref.shape[0]
    C = o_ref.shape[3]

    ones_col = jnp.ones((n, 1), jnp.float32)
    for tt in range(t):
        xb = jnp.concatenate([x_ref[0, :, tt, :], ones_col], axis=1)
        sdall = jnp.dot(xb, wa_ref[...],
                        preferred_element_type=jnp.float32)
        sdT = sdall.T
        U = jnp.dot(xb, wg_ref[...],
                    preferred_element_type=jnp.float32)

        fc = fcb_ref[0][None, :]
        for h in range(H):
            sd = sdall[:, h:h + 1] + sdT[H + h:H + h + 1, :]
            e = jnp.maximum(sd, 0.2 * sd) + bias_ref[h]
            p = jnp.exp(e)
            pu = jnp.dot(p, U[:, h * 128:(h + 1) * 128],
                         preferred_element_type=jnp.float32)
            fc = fc + pu[:, :C] * (1.0 / pu[:, C:C + 1])
        o_ref[0, :, tt, :] = jnp.maximum(fc, 0.0)


def kernel(x, NATree, W, a_src, a_dst, fc_w, fc_b):
    b, n, t, c = x.shape
    H = a_src.shape[0]
    C = W.shape[1] // H

    bias, WA, WG = pl.pallas_call(
        _prep_kernel,
        out_shape=[
            jax.ShapeDtypeStruct((H, n, n), jnp.float32),
            jax.ShapeDtypeStruct((c + 1, 2 * H), jnp.float32),
            jax.ShapeDtypeStruct((c + 1, H * 128), jnp.float32),
        ],
    )(NATree, W, a_src, a_dst, fc_w)

    out = pl.pallas_call(
        _attn_kernel,
        grid=(b,),
        in_specs=[
            pl.BlockSpec((1, n, t, c), lambda i: (i, 0, 0, 0)),
            pl.BlockSpec((H, n, n), lambda i: (0, 0, 0)),
            pl.BlockSpec((c + 1, 2 * H), lambda i: (0, 0)),
            pl.BlockSpec((c + 1, H * 128), lambda i: (0, 0)),
            pl.BlockSpec((1, C), lambda i: (0, 0)),
        ],
        out_specs=pl.BlockSpec((1, n, t, C), lambda i: (i, 0, 0, 0)),
        out_shape=jax.ShapeDtypeStruct((b, n, t, C), jnp.float32),
        compiler_params=pltpu.CompilerParams(
            dimension_semantics=("parallel",),
        ),
    )(x, bias, WA, WG, fc_b.reshape(1, C))
    return out
